# baseline (speedup 1.0000x reference)
import functools

import jax
import jax.numpy as jnp
from jax import lax
from jax.experimental import pallas as pl
from jax.experimental.pallas import tpu as pltpu
from jax.experimental.pallas import tpu_sc as plsc

N = 10000
E = 320000
D = 128
DP = D // 2
L = 16
NC = 2
NS = 16
NW = NC * NS
NP = NS * 640
RPT = NP // NS
EPW = E // NW
C = 80
NCHUNK = EPW // C
KSUP = 25
NSUP = NCHUNK // KSUP
GROUPS = C // L


def _sc_body(x_hbm, src_hbm, dst_hbm, vals_hbm, p_hbm, deg_hbm,
             src_v, dst_v, vals_v, g0_v, g1_v, f0_v, f1_v,
             zdeg_v, acc_sh, dacc_sh,
             gsem0, gsem1, ssem0, ssem1, dsem):
    cid = lax.axis_index("c")
    sid = lax.axis_index("s")
    wid = sid * NC + cid

    zeros16 = jnp.zeros((L,), jnp.float32)

    def zrow(r, carry):
        for j in range(D // L):
            f0_v[r, pl.ds(j * L, L)] = zeros16
        return carry

    lax.fori_loop(0, C, zrow, 0)
    for j in range(RPT // L):
        zdeg_v[pl.ds(j * L, L)] = zeros16

    row0 = pl.multiple_of(sid * RPT, 8)
    for k in range(RPT // C):
        pltpu.sync_copy(f0_v, acc_sh.at[pl.ds(row0 + k * C, C)])
    pltpu.sync_copy(zdeg_v, dacc_sh.at[pl.ds(row0, RPT)])
    plsc.subcore_barrier()

    sh16 = jnp.full((L,), 16, jnp.int32)
    msk16 = jnp.full((L,), -65536, jnp.int32)

    def scale_expand(i, g_v, f_v):
        def group_body(g, carry2):
            vv = vals_v[i, pl.ds(g * L, L)]
            for r in range(L):
                s = vv.at[jnp.full((L,), r, jnp.int32)].get(
                    mode="promise_in_bounds")
                row = g * L + r
                for j in range(D // 32):
                    pr = g_v[row, pl.ds(j * L, L)]
                    lo = plsc.bitcast(lax.shift_left(pr, sh16), jnp.float32)
                    hi = plsc.bitcast(lax.bitwise_and(pr, msk16),
                                      jnp.float32)
                    f_v[row, pl.ds(j * 2 * L, L)] = lo * s
                    f_v[row, pl.ds(j * 2 * L + L, L)] = hi * s
            return carry2

        lax.fori_loop(0, GROUPS, group_body, 0)

    def step(i, g_cur, f_cur, gsem_cur, ssem_cur, g_nxt, gsem_nxt):
        @pl.when(i < KSUP - 1)
        def _():
            pltpu.async_copy(x_hbm.at[src_v.at[i + 1]], g_nxt, gsem_nxt)

        pltpu.make_async_copy(
            x_hbm.at[src_v.at[i]], g_cur, gsem_cur).wait()

        @pl.when(i >= 2)
        def _():
            pltpu.make_async_copy(
                f_cur, acc_sh.at[pl.ds(0, C)], ssem_cur).wait()

        scale_expand(i, g_cur, f_cur)

        pltpu.async_copy(f_cur, acc_sh.at[dst_v.at[i]], ssem_cur, add=True)
        pltpu.async_copy(vals_v.at[i], dacc_sh.at[dst_v.at[i]], dsem, add=True)

    def super_body(sbi, carry):
        pltpu.sync_copy(src_hbm.at[wid, sbi], src_v)
        pltpu.sync_copy(dst_hbm.at[wid, sbi], dst_v)
        pltpu.sync_copy(vals_hbm.at[wid, sbi], vals_v)

        pltpu.async_copy(x_hbm.at[src_v.at[0]], g0_v, gsem0)

        rings = [(g0_v, f0_v, gsem0, ssem0), (g1_v, f1_v, gsem1, ssem1)]

        def chunk_iter(i, carry1):
            for p in range(2):
                @pl.when(i % 2 == p)
                def _(p=p):
                    gc, fc, gs, ss = rings[p]
                    gn, _, gsn, _ = rings[(p + 1) % 2]
                    step(i, gc, fc, gs, ss, gn, gsn)

            return carry1

        lax.fori_loop(0, KSUP, chunk_iter, 0)

        for i in (KSUP - 2, KSUP - 1):
            _, fv, _, sv = rings[i % 2]
            pltpu.make_async_copy(fv, acc_sh.at[pl.ds(0, C)], sv).wait()

        def drain_deg(i, carry2):
            pltpu.make_async_copy(
                vals_v.at[0], dacc_sh.at[dst_v.at[0]], dsem).wait()
            return carry2

        lax.fori_loop(0, KSUP, drain_deg, 0)
        return carry

    lax.fori_loop(0, NSUP, super_body, 0)
    plsc.subcore_barrier()

    pltpu.sync_copy(acc_sh.at[pl.ds(row0, RPT)], p_hbm.at[cid, pl.ds(row0, RPT)])
    pltpu.sync_copy(dacc_sh.at[pl.ds(row0, RPT)], deg_hbm.at[cid, pl.ds(row0, RPT)])


_sc_scatter = functools.partial(
    pl.kernel,
    out_type=[
        jax.ShapeDtypeStruct((NC, NP, D), jnp.float32),
        jax.ShapeDtypeStruct((NC, NP), jnp.float32),
    ],
    mesh=plsc.VectorSubcoreMesh(core_axis_name="c", subcore_axis_name="s"),
    compiler_params=pltpu.CompilerParams(use_tc_tiling_on_sc=False,
                                         needs_layout_passes=False),
    scratch_types=[
        pltpu.VMEM((KSUP, C), jnp.int32),
        pltpu.VMEM((KSUP, C), jnp.int32),
        pltpu.VMEM((KSUP, C), jnp.float32),
        pltpu.VMEM((C, DP), jnp.int32),
        pltpu.VMEM((C, DP), jnp.int32),
        pltpu.VMEM((C, D), jnp.float32),
        pltpu.VMEM((C, D), jnp.float32),
        pltpu.VMEM((RPT,), jnp.float32),
        pltpu.VMEM_SHARED((NP, D), jnp.float32),
        pltpu.VMEM_SHARED((NP,), jnp.float32),
        pltpu.SemaphoreType.DMA,
        pltpu.SemaphoreType.DMA,
        pltpu.SemaphoreType.DMA,
        pltpu.SemaphoreType.DMA,
        pltpu.SemaphoreType.DMA,
    ],
)(_sc_body)


def _mm_body(p0_ref, p1_ref, d0_ref, d1_ref, wt_ref, b_ref, o_ref):
    h = p0_ref[...] + p1_ref[...]
    dd = d0_ref[...] + d1_ref[...]
    o_ref[...] = (jnp.dot(h, wt_ref[...], preferred_element_type=jnp.float32)
                  + dd * b_ref[...])


_R = 2048


def _tc_matmul(p0, p1, d0, d1, wt, b2):
    return pl.pallas_call(
        _mm_body,
        grid=(NP // _R,),
        in_specs=[
            pl.BlockSpec((_R, D), lambda i: (i, 0)),
            pl.BlockSpec((_R, D), lambda i: (i, 0)),
            pl.BlockSpec((_R, 1), lambda i: (i, 0)),
            pl.BlockSpec((_R, 1), lambda i: (i, 0)),
            pl.BlockSpec((D, D), lambda i: (0, 0)),
            pl.BlockSpec((1, D), lambda i: (0, 0)),
        ],
        out_specs=pl.BlockSpec((_R, D), lambda i: (i, 0)),
        out_shape=jax.ShapeDtypeStruct((NP, D), jnp.float32),
    )(p0, p1, d0, d1, wt, b2)


def kernel(x, edge_index, adj_vals, W, b):
    src = edge_index[1].astype(jnp.int32).reshape(NW, NSUP, KSUP, C)
    dst = edge_index[0].astype(jnp.int32).reshape(NW, NSUP, KSUP, C)
    vals = adj_vals.reshape(NW, NSUP, KSUP, C)
    xq = (x.astype(jnp.bfloat16).reshape(N, 4, 2, L)
          .transpose(0, 1, 3, 2).reshape(N, DP, 2))
    xqi = jax.lax.bitcast_convert_type(xq, jnp.int32)
    P, deg = _sc_scatter(xqi, src, dst, vals)
    out = _tc_matmul(P[0], P[1], deg[0][:, None], deg[1][:, None],
                     W.T, b[None, :])
    return out[:N]


# speedup vs baseline: 1.6498x; 1.6498x over previous
import functools

import jax
import jax.numpy as jnp
from jax import lax
from jax.experimental import pallas as pl
from jax.experimental.pallas import tpu as pltpu
from jax.experimental.pallas import tpu_sc as plsc

N = 10000
E = 320000
D = 128
L = 16
NC = 2
NS = 16
NW = NC * NS
NP = NS * 640
RPT = NP // NS
EPW = E // NW
C = 80
NCHUNK = EPW // C
KSUP = 25
NSUP = NCHUNK // KSUP
GROUPS = C // L


def _scale_rows(rows_v, vals_v, i):
    def group_body(g, carry2):
        vv = vals_v[i, pl.ds(g * L, L)]
        for r in range(L):
            s = vv.at[jnp.full((L,), r, jnp.int32)].get(
                mode="promise_in_bounds")
            row = g * L + r
            for j in range(D // L):
                sl = pl.ds(j * L, L)
                rows_v[row, sl] = rows_v[row, sl] * s
        return carry2

    lax.fori_loop(0, GROUPS, group_body, 0)


def _sc_body(x_hbm, src_hbm, dst_hbm, vals_hbm, p_hbm, deg_hbm,
             src_v, dst_v, vals_v, rows0_v, rows1_v, rows2_v, zdeg_v,
             acc_sh, dacc_sh, gsem0, gsem1, gsem2, ssem0, ssem1, ssem2, dsem):
    cid = lax.axis_index("c")
    sid = lax.axis_index("s")
    wid = sid * NC + cid

    zeros16 = jnp.zeros((L,), jnp.float32)

    def zrow(r, carry):
        for j in range(D // L):
            rows0_v[r, pl.ds(j * L, L)] = zeros16
        return carry

    lax.fori_loop(0, C, zrow, 0)
    for j in range(RPT // L):
        zdeg_v[pl.ds(j * L, L)] = zeros16

    row0 = pl.multiple_of(sid * RPT, 8)
    for k in range(RPT // C):
        pltpu.sync_copy(rows0_v, acc_sh.at[pl.ds(row0 + k * C, C)])
    pltpu.sync_copy(zdeg_v, dacc_sh.at[pl.ds(row0, RPT)])
    plsc.subcore_barrier()

    def step(i, cur_rows, cur_gsem, cur_ssem, nxt_rows, nxt_gsem, nxt_ssem):
        @pl.when(i >= 2)
        def _():
            pltpu.make_async_copy(
                nxt_rows, acc_sh.at[pl.ds(0, C)], nxt_ssem).wait()

        @pl.when(i < KSUP - 1)
        def _():
            pltpu.async_copy(x_hbm.at[src_v.at[i + 1]], nxt_rows, nxt_gsem)

        pltpu.make_async_copy(
            x_hbm.at[src_v.at[i]], cur_rows, cur_gsem).wait()

        _scale_rows(cur_rows, vals_v, i)

        pltpu.async_copy(cur_rows, acc_sh.at[dst_v.at[i]], cur_ssem, add=True)
        pltpu.async_copy(vals_v.at[i], dacc_sh.at[dst_v.at[i]], dsem, add=True)

    def super_body(sbi, carry):
        pltpu.sync_copy(src_hbm.at[wid, sbi], src_v)
        pltpu.sync_copy(dst_hbm.at[wid, sbi], dst_v)
        pltpu.sync_copy(vals_hbm.at[wid, sbi], vals_v)

        pltpu.async_copy(x_hbm.at[src_v.at[0]], rows0_v, gsem0)

        rings = [(rows0_v, gsem0, ssem0), (rows1_v, gsem1, ssem1),
                 (rows2_v, gsem2, ssem2)]

        def chunk_iter(i, carry1):
            for p in range(3):
                @pl.when(i % 3 == p)
                def _(p=p):
                    cur = rings[p]
                    nxt = rings[(p + 1) % 3]
                    step(i, *cur, *nxt)

            return carry1

        lax.fori_loop(0, KSUP, chunk_iter, 0)

        for i in (KSUP - 2, KSUP - 1):
            rv, _, sv = rings[i % 3]
            pltpu.make_async_copy(rv, acc_sh.at[pl.ds(0, C)], sv).wait()

        def drain_deg(i, carry2):
            pltpu.make_async_copy(
                vals_v.at[0], dacc_sh.at[dst_v.at[0]], dsem).wait()
            return carry2

        lax.fori_loop(0, KSUP, drain_deg, 0)
        return carry

    lax.fori_loop(0, NSUP, super_body, 0)
    plsc.subcore_barrier()

    pltpu.sync_copy(acc_sh.at[pl.ds(row0, RPT)], p_hbm.at[cid, pl.ds(row0, RPT)])
    pltpu.sync_copy(dacc_sh.at[pl.ds(row0, RPT)], deg_hbm.at[cid, pl.ds(row0, RPT)])


_sc_scatter = functools.partial(
    pl.kernel,
    out_type=[
        jax.ShapeDtypeStruct((NC, NP, D), jnp.float32),
        jax.ShapeDtypeStruct((NC, NP), jnp.float32),
    ],
    mesh=plsc.VectorSubcoreMesh(core_axis_name="c", subcore_axis_name="s"),
    scratch_types=[
        pltpu.VMEM((KSUP, C), jnp.int32),
        pltpu.VMEM((KSUP, C), jnp.int32),
        pltpu.VMEM((KSUP, C), jnp.float32),
        pltpu.VMEM((C, D), jnp.float32),
        pltpu.VMEM((C, D), jnp.float32),
        pltpu.VMEM((C, D), jnp.float32),
        pltpu.VMEM((RPT,), jnp.float32),
        pltpu.VMEM_SHARED((NP, D), jnp.float32),
        pltpu.VMEM_SHARED((NP,), jnp.float32),
        pltpu.SemaphoreType.DMA,
        pltpu.SemaphoreType.DMA,
        pltpu.SemaphoreType.DMA,
        pltpu.SemaphoreType.DMA,
        pltpu.SemaphoreType.DMA,
        pltpu.SemaphoreType.DMA,
        pltpu.SemaphoreType.DMA,
    ],
)(_sc_body)


def _mm_body(p0_ref, p1_ref, d0_ref, d1_ref, wt_ref, b_ref, o_ref):
    h = p0_ref[...] + p1_ref[...]
    dd = d0_ref[...] + d1_ref[...]
    o_ref[...] = (jnp.dot(h, wt_ref[...], preferred_element_type=jnp.float32)
                  + dd * b_ref[...])


_R = 2048


def _tc_matmul(p0, p1, d0, d1, wt, b2):
    return pl.pallas_call(
        _mm_body,
        grid=(NP // _R,),
        in_specs=[
            pl.BlockSpec((_R, D), lambda i: (i, 0)),
            pl.BlockSpec((_R, D), lambda i: (i, 0)),
            pl.BlockSpec((_R, 1), lambda i: (i, 0)),
            pl.BlockSpec((_R, 1), lambda i: (i, 0)),
            pl.BlockSpec((D, D), lambda i: (0, 0)),
            pl.BlockSpec((1, D), lambda i: (0, 0)),
        ],
        out_specs=pl.BlockSpec((_R, D), lambda i: (i, 0)),
        out_shape=jax.ShapeDtypeStruct((NP, D), jnp.float32),
    )(p0, p1, d0, d1, wt, b2)


def kernel(x, edge_index, adj_vals, W, b):
    src = edge_index[1].astype(jnp.int32).reshape(NW, NSUP, KSUP, C)
    dst = edge_index[0].astype(jnp.int32).reshape(NW, NSUP, KSUP, C)
    vals = adj_vals.reshape(NW, NSUP, KSUP, C)
    P, deg = _sc_scatter(x, src, dst, vals)
    out = _tc_matmul(P[0], P[1], deg[0][:, None], deg[1][:, None],
                     W.T, b[None, :])
    return out[:N]


# speedup vs baseline: 1.6615x; 1.0071x over previous
import functools

import jax
import jax.numpy as jnp
from jax import lax
from jax.experimental import pallas as pl
from jax.experimental.pallas import tpu as pltpu
from jax.experimental.pallas import tpu_sc as plsc

N = 10000
E = 320000
D = 128
L = 16
NC = 2
NS = 16
NW = NC * NS
NP = NS * 640
RPT = NP // NS
EPW = E // NW
C = 80
NCHUNK = EPW // C
KSUP = 25
NSUP = NCHUNK // KSUP
GROUPS = C // L


def _scale_rows(rows_v, vals_v, i):
    def group_body(g, carry2):
        vv = vals_v[i, pl.ds(g * L, L)]
        for r in range(L):
            s = vv.at[jnp.full((L,), r, jnp.int32)].get(
                mode="promise_in_bounds")
            row = g * L + r
            for j in range(D // L):
                sl = pl.ds(j * L, L)
                rows_v[row, sl] = rows_v[row, sl] * s
        return carry2

    lax.fori_loop(0, GROUPS, group_body, 0)


def _sc_body(x_hbm, src_hbm, dst_hbm, vals_hbm, p_hbm, deg_hbm,
             src_v, dst_v, vals_v, rows0_v, rows1_v, rows2_v, zdeg_v,
             acc_sh, dacc_sh, gsem0, gsem1, gsem2, ssem0, ssem1, ssem2, dsem):
    cid = lax.axis_index("c")
    sid = lax.axis_index("s")
    wid = sid * NC + cid

    zeros16 = jnp.zeros((L,), jnp.float32)

    def zrow(r, carry):
        for j in range(D // L):
            rows0_v[r, pl.ds(j * L, L)] = zeros16
        return carry

    lax.fori_loop(0, C, zrow, 0)
    for j in range(RPT // L):
        zdeg_v[pl.ds(j * L, L)] = zeros16

    row0 = pl.multiple_of(sid * RPT, 8)
    for k in range(RPT // C):
        pltpu.async_copy(rows0_v, acc_sh.at[pl.ds(row0 + k * C, C)], gsem0)
    pltpu.async_copy(zdeg_v, dacc_sh.at[pl.ds(row0, RPT)], gsem1)
    for k in range(RPT // C):
        pltpu.make_async_copy(
            rows0_v, acc_sh.at[pl.ds(row0 + k * C, C)], gsem0).wait()
    pltpu.make_async_copy(zdeg_v, dacc_sh.at[pl.ds(row0, RPT)], gsem1).wait()
    plsc.subcore_barrier()

    def step(i, cur_rows, cur_gsem, cur_ssem, nxt_rows, nxt_gsem, nxt_ssem):
        @pl.when(i >= 2)
        def _():
            pltpu.make_async_copy(
                nxt_rows, acc_sh.at[pl.ds(0, C)], nxt_ssem).wait()

        @pl.when(i < KSUP - 1)
        def _():
            pltpu.async_copy(x_hbm.at[src_v.at[i + 1]], nxt_rows, nxt_gsem)

        pltpu.make_async_copy(
            x_hbm.at[src_v.at[i]], cur_rows, cur_gsem).wait()

        _scale_rows(cur_rows, vals_v, i)

        pltpu.async_copy(cur_rows, acc_sh.at[dst_v.at[i]], cur_ssem, add=True)
        pltpu.async_copy(vals_v.at[i], dacc_sh.at[dst_v.at[i]], dsem, add=True)

    def super_body(sbi, carry):
        pltpu.sync_copy(src_hbm.at[wid, sbi], src_v)
        pltpu.sync_copy(dst_hbm.at[wid, sbi], dst_v)
        pltpu.sync_copy(vals_hbm.at[wid, sbi], vals_v)

        pltpu.async_copy(x_hbm.at[src_v.at[0]], rows0_v, gsem0)

        rings = [(rows0_v, gsem0, ssem0), (rows1_v, gsem1, ssem1),
                 (rows2_v, gsem2, ssem2)]

        def chunk_iter(i, carry1):
            for p in range(3):
                @pl.when(i % 3 == p)
                def _(p=p):
                    cur = rings[p]
                    nxt = rings[(p + 1) % 3]
                    step(i, *cur, *nxt)

            return carry1

        lax.fori_loop(0, KSUP, chunk_iter, 0)

        for i in (KSUP - 2, KSUP - 1):
            rv, _, sv = rings[i % 3]
            pltpu.make_async_copy(rv, acc_sh.at[pl.ds(0, C)], sv).wait()

        def drain_deg(i, carry2):
            pltpu.make_async_copy(
                vals_v.at[0], dacc_sh.at[dst_v.at[0]], dsem).wait()
            return carry2

        lax.fori_loop(0, KSUP, drain_deg, 0)
        return carry

    lax.fori_loop(0, NSUP, super_body, 0)
    plsc.subcore_barrier()

    pltpu.sync_copy(acc_sh.at[pl.ds(row0, RPT)], p_hbm.at[cid, pl.ds(row0, RPT)])
    pltpu.sync_copy(dacc_sh.at[pl.ds(row0, RPT)], deg_hbm.at[cid, pl.ds(row0, RPT)])


_sc_scatter = functools.partial(
    pl.kernel,
    out_type=[
        jax.ShapeDtypeStruct((NC, NP, D), jnp.float32),
        jax.ShapeDtypeStruct((NC, NP), jnp.float32),
    ],
    mesh=plsc.VectorSubcoreMesh(core_axis_name="c", subcore_axis_name="s"),
    scratch_types=[
        pltpu.VMEM((KSUP, C), jnp.int32),
        pltpu.VMEM((KSUP, C), jnp.int32),
        pltpu.VMEM((KSUP, C), jnp.float32),
        pltpu.VMEM((C, D), jnp.float32),
        pltpu.VMEM((C, D), jnp.float32),
        pltpu.VMEM((C, D), jnp.float32),
        pltpu.VMEM((RPT,), jnp.float32),
        pltpu.VMEM_SHARED((NP, D), jnp.float32),
        pltpu.VMEM_SHARED((NP,), jnp.float32),
        pltpu.SemaphoreType.DMA,
        pltpu.SemaphoreType.DMA,
        pltpu.SemaphoreType.DMA,
        pltpu.SemaphoreType.DMA,
        pltpu.SemaphoreType.DMA,
        pltpu.SemaphoreType.DMA,
        pltpu.SemaphoreType.DMA,
    ],
)(_sc_body)


def _mm_body(p0_ref, p1_ref, d0_ref, d1_ref, wt_ref, b_ref, o_ref):
    h = p0_ref[...] + p1_ref[...]
    dd = d0_ref[...] + d1_ref[...]
    o_ref[...] = (jnp.dot(h, wt_ref[...], preferred_element_type=jnp.float32)
                  + dd * b_ref[...])


_R = 2048


def _tc_matmul(p0, p1, d0, d1, wt, b2):
    return pl.pallas_call(
        _mm_body,
        grid=(NP // _R,),
        in_specs=[
            pl.BlockSpec((_R, D), lambda i: (i, 0)),
            pl.BlockSpec((_R, D), lambda i: (i, 0)),
            pl.BlockSpec((_R, 1), lambda i: (i, 0)),
            pl.BlockSpec((_R, 1), lambda i: (i, 0)),
            pl.BlockSpec((D, D), lambda i: (0, 0)),
            pl.BlockSpec((1, D), lambda i: (0, 0)),
        ],
        out_specs=pl.BlockSpec((_R, D), lambda i: (i, 0)),
        out_shape=jax.ShapeDtypeStruct((NP, D), jnp.float32),
    )(p0, p1, d0, d1, wt, b2)


def kernel(x, edge_index, adj_vals, W, b):
    src = edge_index[1].astype(jnp.int32).reshape(NW, NSUP, KSUP, C)
    dst = edge_index[0].astype(jnp.int32).reshape(NW, NSUP, KSUP, C)
    vals = adj_vals.reshape(NW, NSUP, KSUP, C)
    P, deg = _sc_scatter(x, src, dst, vals)
    out = _tc_matmul(P[0], P[1], deg[0][:, None], deg[1][:, None],
                     W.T, b[None, :])
    return out[:N]
